# Initial kernel scaffold; baseline (speedup 1.0000x reference)
import jax
import jax.numpy as jnp
from jax.experimental import pallas as pl


def kernel(x, adj, W1, b1, W2, b2, W3, b3):
    raise NotImplementedError("write your pallas kernel here")


# speedup vs baseline: 10.6625x; 10.6625x over previous
import functools

import jax
import jax.numpy as jnp
from jax import lax
from jax.experimental import pallas as pl
from jax.experimental.pallas import tpu as pltpu
from jax.experimental.pallas import tpu_sc as plsc

N = 10000
E = 320000
FEAT = 128
NCLS = 64

NC = 2
NS = 16
NW = NC * NS
EPW = E // NW
K = 80
NCHUNK = EPW // K
NPAD = 10240
NPS = NPAD // NS

_MESH = plsc.VectorSubcoreMesh(core_axis_name="c", subcore_axis_name="s")


@functools.partial(
    pl.kernel,
    out_type=jax.ShapeDtypeStruct((NC, NPAD, FEAT), jnp.float32),
    mesh=_MESH,
    scratch_types=[
        pltpu.VMEM((K,), jnp.int32),
        pltpu.VMEM((K, FEAT), jnp.float32),
        pltpu.VMEM_SHARED((NPAD, FEAT), jnp.float32),
    ],
)
def _deg_kernel(dst_hbm, ones_hbm, zeros_hbm, out_hbm, didx, ones_v, hist):
    c = lax.axis_index("c")
    s = lax.axis_index("s")
    wid = c * NS + s
    pltpu.sync_copy(ones_hbm, ones_v)
    pltpu.sync_copy(zeros_hbm.at[pl.ds(s * NPS, NPS)],
                    hist.at[pl.ds(s * NPS, NPS)])
    plsc.subcore_barrier()

    def body(j, carry):
        off = wid * EPW + j * K
        pltpu.sync_copy(dst_hbm.at[pl.ds(off, K)], didx)
        pltpu.sync_copy(ones_v, hist.at[didx], add=True)
        return carry

    lax.fori_loop(0, NCHUNK, body, 0)
    plsc.subcore_barrier()
    pltpu.sync_copy(hist.at[pl.ds(s * NPS, NPS)],
                    out_hbm.at[c, pl.ds(s * NPS, NPS)])


def _make_prop(feat):
    @functools.partial(
        pl.kernel,
        out_type=jax.ShapeDtypeStruct((NC, NPAD, feat), jnp.float32),
        mesh=_MESH,
        scratch_types=[
            pltpu.VMEM((K,), jnp.int32),
            pltpu.VMEM((K,), jnp.int32),
            pltpu.VMEM((K, feat), jnp.float32),
            pltpu.VMEM_SHARED((NPAD, feat), jnp.float32),
            pltpu.SemaphoreType.DMA,
        ],
    )
    def prop(y_hbm, src_hbm, dst_hbm, zeros_hbm, out_hbm,
             sidx, didx, rows, acc, sem):
        c = lax.axis_index("c")
        s = lax.axis_index("s")
        wid = c * NS + s
        base = wid * EPW
        pltpu.sync_copy(zeros_hbm.at[pl.ds(s * NPS, NPS)],
                        acc.at[pl.ds(s * NPS, NPS)])
        plsc.subcore_barrier()

        def body(j, carry):
            off = base + j * K
            pltpu.sync_copy(src_hbm.at[pl.ds(off, K)], sidx)
            pltpu.sync_copy(dst_hbm.at[pl.ds(off, K)], didx)
            pltpu.async_copy(y_hbm.at[sidx], rows, sem).wait()
            pltpu.sync_copy(rows, acc.at[didx], add=True)
            return carry

        lax.fori_loop(0, NCHUNK, body, 0)
        plsc.subcore_barrier()
        pltpu.sync_copy(acc.at[pl.ds(s * NPS, NPS)],
                        out_hbm.at[c, pl.ds(s * NPS, NPS)])

    return prop


_prop128 = _make_prop(FEAT)


_BR = 1000


def _tc_first(x, W1, cnt):
    def body(x_ref, w_ref, c_ref, y_ref, dv_ref):
        h = jnp.dot(x_ref[...], w_ref[...], preferred_element_type=jnp.float32)
        deg = c_ref[0, :, 0] + c_ref[1, :, 0] + 1.0
        dv = lax.rsqrt(deg)
        y_ref[...] = h * dv[:, None]
        dv_ref[...] = dv[:, None]

    return pl.pallas_call(
        body,
        grid=(N // _BR,),
        in_specs=[
            pl.BlockSpec((_BR, FEAT), lambda i: (i, 0)),
            pl.BlockSpec((FEAT, FEAT), lambda i: (0, 0)),
            pl.BlockSpec((NC, _BR, FEAT), lambda i: (0, i, 0)),
        ],
        out_specs=[
            pl.BlockSpec((_BR, FEAT), lambda i: (i, 0)),
            pl.BlockSpec((_BR, 1), lambda i: (i, 0)),
        ],
        out_shape=[
            jax.ShapeDtypeStruct((N, FEAT), jnp.float32),
            jax.ShapeDtypeStruct((N, 1), jnp.float32),
        ],
    )(x, W1, cnt)


def _tc_mid(p, y, dinv, b, W, fin, fout):
    def body(p_ref, y_ref, dv_ref, b_ref, w_ref, o_ref):
        t = dv_ref[...] * (p_ref[0] + p_ref[1] + y_ref[...]) + b_ref[...]
        a = jnp.tanh(t)
        o_ref[...] = dv_ref[...] * jnp.dot(
            a, w_ref[...], preferred_element_type=jnp.float32)

    return pl.pallas_call(
        body,
        grid=(N // _BR,),
        in_specs=[
            pl.BlockSpec((NC, _BR, fin), lambda i: (0, i, 0)),
            pl.BlockSpec((_BR, fin), lambda i: (i, 0)),
            pl.BlockSpec((_BR, 1), lambda i: (i, 0)),
            pl.BlockSpec((1, fin), lambda i: (0, 0)),
            pl.BlockSpec((fin, fout), lambda i: (0, 0)),
        ],
        out_specs=pl.BlockSpec((_BR, fout), lambda i: (i, 0)),
        out_shape=jax.ShapeDtypeStruct((N, fout), jnp.float32),
    )(p, y, dinv, b, W)


def _tc_last(p, y, dinv, b):
    def body(p_ref, y_ref, dv_ref, b_ref, o_ref):
        t = dv_ref[...] * (p_ref[0] + p_ref[1] + y_ref[...])
        o_ref[...] = t[:, :NCLS] + b_ref[...]

    return pl.pallas_call(
        body,
        grid=(N // _BR,),
        in_specs=[
            pl.BlockSpec((NC, _BR, FEAT), lambda i: (0, i, 0)),
            pl.BlockSpec((_BR, FEAT), lambda i: (i, 0)),
            pl.BlockSpec((_BR, 1), lambda i: (i, 0)),
            pl.BlockSpec((1, NCLS), lambda i: (0, 0)),
        ],
        out_specs=pl.BlockSpec((_BR, NCLS), lambda i: (i, 0)),
        out_shape=jax.ShapeDtypeStruct((N, NCLS), jnp.float32),
    )(p, y, dinv, b)


def kernel(x, adj, W1, b1, W2, b2, W3, b3):
    adj = adj.astype(jnp.int32)
    src = adj[0]
    dst = adj[1]
    ones128 = jnp.ones((K, FEAT), jnp.float32)
    zeros128 = jnp.zeros((NPAD, FEAT), jnp.float32)
    W3p = jnp.pad(W3, ((0, 0), (0, FEAT - NCLS)))

    cnt = _deg_kernel(dst, ones128, zeros128)
    y1, dinv = _tc_first(x, W1, cnt)
    p = _prop128(y1, src, dst, zeros128)
    y2 = _tc_mid(p, y1, dinv, b1.reshape(1, -1), W2, FEAT, FEAT)
    p = _prop128(y2, src, dst, zeros128)
    y3 = _tc_mid(p, y2, dinv, b2.reshape(1, -1), W3p, FEAT, FEAT)
    p = _prop128(y3, src, dst, zeros128)
    return _tc_last(p, y3, dinv, b3.reshape(1, -1))
